# Initial kernel scaffold; baseline (speedup 1.0000x reference)
import jax
import jax.numpy as jnp
from jax.experimental import pallas as pl


def kernel(input, input1):
    raise NotImplementedError("write your pallas kernel here")


# speedup vs baseline: 1.7902x; 1.7902x over previous
import functools

import jax
import jax.numpy as jnp
from jax import lax
from jax.experimental import pallas as pl
from jax.experimental.pallas import tpu as pltpu
from jax.experimental.pallas import tpu_sc as plsc

L = 16
PP = 32
JP = 32
SLOTS = PP * JP
GCH = SLOTS // 128


def _bc(s):
    return jnp.broadcast_to(s, (L,))


@functools.lru_cache(maxsize=None)
def _build(B, N, P, J):
    mesh = plsc.VectorSubcoreMesh(core_axis_name="c", subcore_axis_name="s")
    NC = 2

    @functools.partial(
        pl.kernel,
        mesh=mesh,
        out_type=jax.ShapeDtypeStruct((B, L), jnp.float32),
        compiler_params=pltpu.CompilerParams(needs_layout_passes=False),
        scratch_types=[
            pltpu.VMEM((GCH, 128), jnp.int32),
            pltpu.VMEM((SLOTS,), jnp.int32),
            pltpu.VMEM((SLOTS,), jnp.float32),
            pltpu.VMEM((L,), jnp.float32),
            pltpu.SemaphoreType.DMA,
        ],
    )
    def aeloss(tags_hbm, idx_hbm, flg_hbm, out_hbm, idx_v, flg_v, val_v, out_v, sem):
        wid = lax.axis_index("s") * NC + lax.axis_index("c")

        pltpu.sync_copy(idx_hbm.at[wid], idx_v)
        pltpu.sync_copy(flg_hbm.at[wid], flg_v)

        off = _bc(wid * N).astype(jnp.int32)
        for j in range(GCH):
            for c in range(128 // L):
                sl = idx_v[j, pl.ds(c * L, L)]
                idx_v[j, pl.ds(c * L, L)] = sl + off

        copies = [
            pltpu.async_copy(
                tags_hbm.at[idx_v.at[j]], val_v.at[pl.ds(j * 128, 128)], sem
            )
            for j in range(GCH)
        ]
        for cp in copies:
            cp.wait()

        zero = jnp.zeros((L,), jnp.float32)
        one = jnp.full((L,), 1.0, jnp.float32)
        lane = lax.iota(jnp.int32, L)

        def person_stats(p):
            base = p * JP
            v0 = val_v[pl.ds(base, L)]
            v1 = val_v[pl.ds(base + L, L)]
            f0 = flg_v[pl.ds(base, L)]
            f1 = flg_v[pl.ds(base + L, L)]
            vis0 = jnp.where(f0 > 0, one, zero)
            vis1 = jnp.where(f1 > 0, one, zero)
            cnt = _bc(jnp.sum(vis0 + vis1))
            safe = jnp.maximum(cnt, one)
            mean = _bc(jnp.sum(v0 * vis0 + v1 * vis1)) / safe
            valid = jnp.where(cnt > 0, one, zero)
            return v0, v1, vis0, vis1, mean, valid, safe

        means_lo = zero
        means_hi = zero
        valid_lo = zero
        valid_hi = zero
        pull_acc = zero
        ntags = zero
        for p in range(P):
            v0, v1, vis0, vis1, mean, valid, safe = person_stats(p)
            d0 = v0 - mean
            d1 = v1 - mean
            pp = _bc(jnp.sum(d0 * d0 * vis0 + d1 * d1 * vis1))
            pull_acc = pull_acc + pp / safe * valid
            ntags = ntags + valid
            if p < L:
                sel = lane == p
                means_lo = jnp.where(sel, mean, means_lo)
                valid_lo = jnp.where(sel, valid, valid_lo)
            else:
                sel = lane == (p - L)
                means_hi = jnp.where(sel, mean, means_hi)
                valid_hi = jnp.where(sel, valid, valid_hi)

        acc_lo = zero
        acc_hi = zero
        for p in range(P):
            _, _, _, _, mean_i, valid_i, _ = person_stats(p)
            dlo = mean_i - means_lo
            dhi = mean_i - means_hi
            acc_lo = acc_lo + valid_i * jnp.exp(-(dlo * dlo)) * valid_lo
            acc_hi = acc_hi + valid_i * jnp.exp(-(dhi * dhi)) * valid_hi

        push_tot = _bc(jnp.sum(acc_lo) + jnp.sum(acc_hi)) - ntags
        denom = jnp.maximum(ntags * (ntags - one), one)
        push = 0.5 * push_tot / denom
        pull = pull_acc / jnp.maximum(ntags, one)

        out_v[...] = jnp.where(lane == 0, pull, jnp.where(lane == 1, push, zero))
        pltpu.sync_copy(out_v, out_hbm.at[wid])

    return aeloss


def kernel(input, input1):
    tags = input
    keypoints = input1
    B, N, D = tags.shape
    P, J = keypoints.shape[1], keypoints.shape[2]

    idx = keypoints[..., 0]
    flg = keypoints[..., 1]
    idx_pad = jnp.zeros((B, PP, JP), jnp.int32).at[:, :P, :J].set(idx)
    flg_pad = jnp.zeros((B, PP, JP), jnp.int32).at[:, :P, :J].set(flg)

    out16 = _build(B, N, P, J)(
        tags.reshape(B * N),
        idx_pad.reshape(B, GCH, 128),
        flg_pad.reshape(B, SLOTS),
    )
    return out16[:, :2]
